# baseline (speedup 1.0000x reference)
import functools

import jax
import jax.numpy as jnp
from jax import lax
from jax.experimental import pallas as pl
from jax.experimental.pallas import tpu as pltpu
from jax.experimental.pallas import tpu_sc as plsc

BATCH = 16384
EMBED = 64
H1, H2, H3 = 128, 64, 32

NUM_CORES = 2
NUM_SUBCORES = 16
NUM_WORKERS = NUM_CORES * NUM_SUBCORES
B_PER_W = BATCH // NUM_WORKERS


def _make_gather():
  mesh = plsc.VectorSubcoreMesh(core_axis_name="c", subcore_axis_name="s")

  @functools.partial(
      pl.kernel,
      mesh=mesh,
      compiler_params=pltpu.CompilerParams(use_tc_tiling_on_sc=False),
      out_type=jax.ShapeDtypeStruct((BATCH, EMBED), jnp.float32),
      scratch_types=[
          pltpu.VMEM((B_PER_W,), jnp.int32),
          pltpu.VMEM((B_PER_W, EMBED), jnp.float32),
          pltpu.SemaphoreType.DMA,
      ],
  )
  def gather(table_hbm, idx_hbm, out_hbm, idx_v, rows_v, sem):
    wid = lax.axis_index("s") * NUM_CORES + lax.axis_index("c")
    base = wid * B_PER_W
    pltpu.sync_copy(idx_hbm.at[pl.ds(base, B_PER_W)], idx_v)
    pltpu.async_copy(table_hbm.at[idx_v], rows_v, sem).wait()
    pltpu.sync_copy(rows_v, out_hbm.at[pl.ds(base, B_PER_W)])

  return gather


_sc_gather = _make_gather()

BLK = 2048


def _mlp_body(emb, w1, b1, w2, b2, w3, b3, out):
  h = jnp.maximum(
      jnp.dot(emb[...], w1[...], preferred_element_type=jnp.float32) + b1[...],
      0.0)
  h = jnp.maximum(
      jnp.dot(h, w2[...], preferred_element_type=jnp.float32) + b2[...], 0.0)
  out[...] = jnp.dot(h, w3[...], preferred_element_type=jnp.float32) + b3[...]


def _mlp(emb, W1, b1, W2, b2, W3, b3):
  grid = (BATCH // BLK,)
  full = lambda shape: pl.BlockSpec(shape, lambda i: (0, 0))
  return pl.pallas_call(
      _mlp_body,
      grid=grid,
      in_specs=[
          pl.BlockSpec((BLK, EMBED), lambda i: (i, 0)),
          full((EMBED, H1)),
          full((1, H1)),
          full((H1, H2)),
          full((1, H2)),
          full((H2, H3)),
          full((1, H3)),
      ],
      out_specs=pl.BlockSpec((BLK, H3), lambda i: (i, 0)),
      out_shape=jax.ShapeDtypeStruct((BATCH, H3), jnp.float32),
  )(emb, W1, b1.reshape(1, H1), W2, b2.reshape(1, H2), W3, b3.reshape(1, H3))


@jax.jit
def kernel(books, table, W1, b1, W2, b2, W3, b3):
  emb = _sc_gather(table, books)
  return _mlp(emb, W1, b1, W2, b2, W3, b3)


# speedup vs baseline: 1.1372x; 1.1372x over previous
import functools

import jax
import jax.numpy as jnp
from jax import lax
from jax.experimental import pallas as pl
from jax.experimental.pallas import tpu as pltpu
from jax.experimental.pallas import tpu_sc as plsc

BATCH = 16384
EMBED = 64
H1, H2, H3 = 128, 64, 32

NUM_CORES = 2
NUM_SUBCORES = 16
NUM_WORKERS = NUM_CORES * NUM_SUBCORES
B_PER_W = BATCH // NUM_WORKERS
CHUNK = 16


def _make_gather():
  mesh = plsc.VectorSubcoreMesh(core_axis_name="c", subcore_axis_name="s")

  @functools.partial(
      pl.kernel,
      mesh=mesh,
      out_type=jax.ShapeDtypeStruct((BATCH, EMBED), jnp.float32),
      scratch_types=[
          pltpu.VMEM((B_PER_W,), jnp.int32),
          pltpu.VMEM((B_PER_W, EMBED), jnp.float32),
          pltpu.SemaphoreType.DMA,
      ],
  )
  def gather(table_hbm, idx_hbm, out_hbm, idx_v, rows_v, sem):
    wid = lax.axis_index("s") * NUM_CORES + lax.axis_index("c")
    base = wid * B_PER_W
    pltpu.sync_copy(idx_hbm.at[pl.ds(base, B_PER_W)], idx_v)

    def chunk_body(c, _):
      cbase = c * CHUNK
      vec = idx_v[pl.ds(cbase, CHUNK)]
      for i in range(CHUNK):
        pltpu.make_async_copy(
            table_hbm.at[vec[i]], rows_v.at[cbase + i], sem).start()
      pltpu.make_async_copy(
          table_hbm.at[pl.ds(0, CHUNK)],
          rows_v.at[pl.ds(cbase, CHUNK)], sem).wait()
      return 0

    lax.fori_loop(0, B_PER_W // CHUNK, chunk_body, 0)
    pltpu.sync_copy(rows_v, out_hbm.at[pl.ds(base, B_PER_W)])

  return gather


_sc_gather = _make_gather()

BLK = 2048


def _mlp_body(emb, w1, b1, w2, b2, w3, b3, out):
  h = jnp.maximum(
      jnp.dot(emb[...], w1[...], preferred_element_type=jnp.float32) + b1[...],
      0.0)
  h = jnp.maximum(
      jnp.dot(h, w2[...], preferred_element_type=jnp.float32) + b2[...], 0.0)
  out[...] = jnp.dot(h, w3[...], preferred_element_type=jnp.float32) + b3[...]


def _mlp(emb, W1, b1, W2, b2, W3, b3):
  grid = (BATCH // BLK,)
  full = lambda shape: pl.BlockSpec(shape, lambda i: (0, 0))
  return pl.pallas_call(
      _mlp_body,
      grid=grid,
      in_specs=[
          pl.BlockSpec((BLK, EMBED), lambda i: (i, 0)),
          full((EMBED, H1)),
          full((1, H1)),
          full((H1, H2)),
          full((1, H2)),
          full((H2, H3)),
          full((1, H3)),
      ],
      out_specs=pl.BlockSpec((BLK, H3), lambda i: (i, 0)),
      out_shape=jax.ShapeDtypeStruct((BATCH, H3), jnp.float32),
  )(emb, W1, b1.reshape(1, H1), W2, b2.reshape(1, H2), W3, b3.reshape(1, H3))


@jax.jit
def kernel(books, table, W1, b1, W2, b2, W3, b3):
  emb = _sc_gather(table, books)
  return _mlp(emb, W1, b1, W2, b2, W3, b3)


# speedup vs baseline: 1.1452x; 1.0070x over previous
import functools

import jax
import jax.numpy as jnp
from jax import lax
from jax.experimental import pallas as pl
from jax.experimental.pallas import tpu as pltpu
from jax.experimental.pallas import tpu_sc as plsc

BATCH = 16384
EMBED = 64
H1, H2, H3 = 128, 64, 32

NUM_CORES = 2
NUM_SUBCORES = 16
NUM_WORKERS = NUM_CORES * NUM_SUBCORES
B_PER_W = BATCH // NUM_WORKERS
CHUNK = 16


def _make_gather():
  mesh = plsc.VectorSubcoreMesh(core_axis_name="c", subcore_axis_name="s")

  @functools.partial(
      pl.kernel,
      mesh=mesh,
      compiler_params=pltpu.CompilerParams(use_tc_tiling_on_sc=True),
      out_type=jax.ShapeDtypeStruct((BATCH, EMBED), jnp.float32),
      scratch_types=[
          pltpu.VMEM((B_PER_W,), jnp.int32),
          pltpu.VMEM((B_PER_W, EMBED), jnp.float32),
          pltpu.SemaphoreType.DMA,
      ],
  )
  def gather(table_hbm, idx_hbm, out_hbm, idx_v, rows_v, sem):
    wid = lax.axis_index("s") * NUM_CORES + lax.axis_index("c")
    base = wid * B_PER_W
    pltpu.sync_copy(idx_hbm.at[pl.ds(base, B_PER_W)], idx_v)

    def chunk_body(c, _):
      cbase = c * CHUNK
      vec = idx_v[pl.ds(cbase, CHUNK)]
      for i in range(CHUNK):
        pltpu.make_async_copy(
            table_hbm.at[vec[i]], rows_v.at[cbase + i], sem).start()
      pltpu.make_async_copy(
          table_hbm.at[pl.ds(0, CHUNK)],
          rows_v.at[pl.ds(cbase, CHUNK)], sem).wait()
      return 0

    lax.fori_loop(0, B_PER_W // CHUNK, chunk_body, 0)
    pltpu.sync_copy(rows_v, out_hbm.at[pl.ds(base, B_PER_W)])

  return gather


_sc_gather = _make_gather()

BLK = 2048


def _mlp_body(emb, w1, b1, w2, b2, w3, b3, out):
  h = jnp.maximum(
      jnp.dot(emb[...], w1[...], preferred_element_type=jnp.float32) + b1[...],
      0.0)
  h = jnp.maximum(
      jnp.dot(h, w2[...], preferred_element_type=jnp.float32) + b2[...], 0.0)
  out[...] = jnp.dot(h, w3[...], preferred_element_type=jnp.float32) + b3[...]


def _mlp(emb, W1, b1, W2, b2, W3, b3):
  grid = (BATCH // BLK,)
  full = lambda shape: pl.BlockSpec(shape, lambda i: (0, 0))
  return pl.pallas_call(
      _mlp_body,
      grid=grid,
      in_specs=[
          pl.BlockSpec((BLK, EMBED), lambda i: (i, 0)),
          full((EMBED, H1)),
          full((1, H1)),
          full((H1, H2)),
          full((1, H2)),
          full((H2, H3)),
          full((1, H3)),
      ],
      out_specs=pl.BlockSpec((BLK, H3), lambda i: (i, 0)),
      out_shape=jax.ShapeDtypeStruct((BATCH, H3), jnp.float32),
  )(emb, W1, b1.reshape(1, H1), W2, b2.reshape(1, H2), W3, b3.reshape(1, H3))


@jax.jit
def kernel(books, table, W1, b1, W2, b2, W3, b3):
  emb = _sc_gather(table, books)
  return _mlp(emb, W1, b1, W2, b2, W3, b3)


# speedup vs baseline: 1.6276x; 1.4212x over previous
import functools

import jax
import jax.numpy as jnp
from jax import lax
from jax.experimental import pallas as pl
from jax.experimental.pallas import tpu as pltpu
from jax.experimental.pallas import tpu_sc as plsc

BATCH = 16384
VOCAB = 100001
EMBED = 64
H1, H2, H3 = 128, 64, 32

NUM_CORES = 2
NUM_SUBCORES = 16
NUM_WORKERS = NUM_CORES * NUM_SUBCORES
ROWS_PER_TILE = EMBED // NUM_WORKERS
IDX_CHUNK = 4096
LANES = 16


def _make_gather():
  mesh = plsc.VectorSubcoreMesh(
      core_axis_name="c", subcore_axis_name="s",
      num_cores=NUM_CORES, num_subcores=NUM_SUBCORES)

  @functools.partial(
      pl.kernel,
      mesh=mesh,
      compiler_params=pltpu.CompilerParams(
          use_tc_tiling_on_sc=True, needs_layout_passes=False),
      out_type=jax.ShapeDtypeStruct((EMBED, BATCH), jnp.float32),
      scratch_types=[
          pltpu.VMEM((VOCAB,), jnp.float32),
          pltpu.VMEM((IDX_CHUNK,), jnp.int32),
          pltpu.VMEM((IDX_CHUNK,), jnp.float32),
      ],
  )
  def gather(tablet_hbm, idx_hbm, out_hbm, row_v, idx_v, out_v):
    tid = lax.axis_index("s") * NUM_CORES + lax.axis_index("c")
    for p in range(ROWS_PER_TILE):
      j = tid * ROWS_PER_TILE + p
      pltpu.sync_copy(tablet_hbm.at[j], row_v)
      for c in range(BATCH // IDX_CHUNK):
        pltpu.sync_copy(idx_hbm.at[pl.ds(c * IDX_CHUNK, IDX_CHUNK)], idx_v)

        def gbody(i, _):
          iv = idx_v[pl.ds(i * LANES, LANES)]
          out_v[pl.ds(i * LANES, LANES)] = plsc.load_gather(row_v, [iv])
          return 0

        lax.fori_loop(0, IDX_CHUNK // LANES, gbody, 0, unroll=8)
        pltpu.sync_copy(out_v,
                        out_hbm.at[j, pl.ds(c * IDX_CHUNK, IDX_CHUNK)])

  return gather


_sc_gather = _make_gather()

BLK = 2048


def _mlp_body(embt, w1t, b1, w2t, b2, w3t, b3, out):
  h = jnp.maximum(
      jnp.dot(w1t[...], embt[...], preferred_element_type=jnp.float32)
      + b1[...], 0.0)
  h = jnp.maximum(
      jnp.dot(w2t[...], h, preferred_element_type=jnp.float32) + b2[...], 0.0)
  out[...] = jnp.dot(w3t[...], h, preferred_element_type=jnp.float32) + b3[...]


def _mlp_t(embt, W1, b1, W2, b2, W3, b3):
  grid = (BATCH // BLK,)
  full = lambda shape: pl.BlockSpec(shape, lambda i: (0, 0))
  return pl.pallas_call(
      _mlp_body,
      grid=grid,
      in_specs=[
          pl.BlockSpec((EMBED, BLK), lambda i: (0, i)),
          full((H1, EMBED)),
          full((H1, 1)),
          full((H2, H1)),
          full((H2, 1)),
          full((H3, H2)),
          full((H3, 1)),
      ],
      out_specs=pl.BlockSpec((H3, BLK), lambda i: (0, i)),
      out_shape=jax.ShapeDtypeStruct((H3, BATCH), jnp.float32),
  )(embt, W1.T, b1.reshape(H1, 1), W2.T, b2.reshape(H2, 1), W3.T,
    b3.reshape(H3, 1))


@jax.jit
def kernel(books, table, W1, b1, W2, b2, W3, b3):
  embt = _sc_gather(table.T, books)
  outt = _mlp_t(embt, W1, b1, W2, b2, W3, b3)
  return outt.T


# speedup vs baseline: 2.0563x; 1.2634x over previous
import functools

import jax
import jax.numpy as jnp
from jax import lax
from jax.experimental import pallas as pl
from jax.experimental.pallas import tpu as pltpu
from jax.experimental.pallas import tpu_sc as plsc

BATCH = 16384
VOCAB = 100001
EMBED = 64
H1, H2, H3 = 128, 64, 32

NUM_CORES = 2
NUM_SUBCORES = 16
NUM_WORKERS = NUM_CORES * NUM_SUBCORES
ROWS_PER_TILE = EMBED // NUM_WORKERS
IDX_CHUNK = 4096
LANES = 16


def _make_gather():
  mesh = plsc.VectorSubcoreMesh(
      core_axis_name="c", subcore_axis_name="s",
      num_cores=NUM_CORES, num_subcores=NUM_SUBCORES)

  @functools.partial(
      pl.kernel,
      mesh=mesh,
      compiler_params=pltpu.CompilerParams(
          use_tc_tiling_on_sc=True, needs_layout_passes=False),
      out_type=jax.ShapeDtypeStruct((EMBED, BATCH), jnp.float32),
      scratch_types=[
          pltpu.VMEM((VOCAB,), jnp.float32),
          pltpu.VMEM((IDX_CHUNK,), jnp.int32),
          pltpu.VMEM((IDX_CHUNK,), jnp.float32),
      ],
  )
  def gather(tablet_hbm, idx_hbm, out_hbm, row_v, idx_v, out_v):
    tid = lax.axis_index("s") * NUM_CORES + lax.axis_index("c")
    for p in range(ROWS_PER_TILE):
      j = tid * ROWS_PER_TILE + p
      pltpu.sync_copy(tablet_hbm.at[j], row_v)
      for c in range(BATCH // IDX_CHUNK):
        pltpu.sync_copy(idx_hbm.at[pl.ds(c * IDX_CHUNK, IDX_CHUNK)], idx_v)

        @plsc.parallel_loop(0, IDX_CHUNK, step=LANES, unroll=8)
        def gbody(i):
          iv = idx_v[pl.ds(i, LANES)]
          out_v[pl.ds(i, LANES)] = plsc.load_gather(row_v, [iv])
        pltpu.sync_copy(out_v,
                        out_hbm.at[j, pl.ds(c * IDX_CHUNK, IDX_CHUNK)])

  return gather


_sc_gather = _make_gather()

BLK = 2048


def _mlp_body(embt, w1t, b1, w2t, b2, w3t, b3, out):
  h = jnp.maximum(
      jnp.dot(w1t[...], embt[...], preferred_element_type=jnp.float32)
      + b1[...], 0.0)
  h = jnp.maximum(
      jnp.dot(w2t[...], h, preferred_element_type=jnp.float32) + b2[...], 0.0)
  out[...] = jnp.dot(w3t[...], h, preferred_element_type=jnp.float32) + b3[...]


def _mlp_t(embt, W1, b1, W2, b2, W3, b3):
  grid = (BATCH // BLK,)
  full = lambda shape: pl.BlockSpec(shape, lambda i: (0, 0))
  return pl.pallas_call(
      _mlp_body,
      grid=grid,
      in_specs=[
          pl.BlockSpec((EMBED, BLK), lambda i: (0, i)),
          full((H1, EMBED)),
          full((H1, 1)),
          full((H2, H1)),
          full((H2, 1)),
          full((H3, H2)),
          full((H3, 1)),
      ],
      out_specs=pl.BlockSpec((H3, BLK), lambda i: (0, i)),
      out_shape=jax.ShapeDtypeStruct((H3, BATCH), jnp.float32),
  )(embt, W1.T, b1.reshape(H1, 1), W2.T, b2.reshape(H2, 1), W3.T,
    b3.reshape(H3, 1))


@jax.jit
def kernel(books, table, W1, b1, W2, b2, W3, b3):
  embt = _sc_gather(table.T, books)
  outt = _mlp_t(embt, W1, b1, W2, b2, W3, b3)
  return outt.T


# speedup vs baseline: 2.3723x; 1.1537x over previous
import functools

import jax
import jax.numpy as jnp
from jax import lax
from jax.experimental import pallas as pl
from jax.experimental.pallas import tpu as pltpu
from jax.experimental.pallas import tpu_sc as plsc

BATCH = 16384
VOCAB = 100001
EMBED = 64
H1, H2, H3 = 128, 64, 32

NUM_CORES = 2
NUM_SUBCORES = 16
NUM_WORKERS = NUM_CORES * NUM_SUBCORES
ROWS_PER_TILE = EMBED // NUM_WORKERS
IDX_CHUNK = 4096
LANES = 16


def _make_gather():
  mesh = plsc.VectorSubcoreMesh(
      core_axis_name="c", subcore_axis_name="s",
      num_cores=NUM_CORES, num_subcores=NUM_SUBCORES)

  @functools.partial(
      pl.kernel,
      mesh=mesh,
      compiler_params=pltpu.CompilerParams(
          use_tc_tiling_on_sc=True, needs_layout_passes=False),
      out_type=jax.ShapeDtypeStruct((EMBED, BATCH), jnp.float32),
      scratch_types=[
          pltpu.VMEM((VOCAB,), jnp.float32),
          pltpu.VMEM((BATCH,), jnp.int32),
          pltpu.VMEM((2, IDX_CHUNK), jnp.float32),
          pltpu.SemaphoreType.DMA,
      ],
  )
  def gather(tablet_hbm, idx_hbm, out_hbm, row_v, idx_v, out_v, osem):
    tid = lax.axis_index("s") * NUM_CORES + lax.axis_index("c")
    pltpu.sync_copy(idx_hbm, idx_v)
    pending = [None, None]
    for p in range(ROWS_PER_TILE):
      j = tid * ROWS_PER_TILE + p
      pltpu.sync_copy(tablet_hbm.at[j], row_v)
      for c in range(BATCH // IDX_CHUNK):
        buf = c % 2
        if pending[buf] is not None:
          pending[buf].wait()

        @plsc.parallel_loop(0, IDX_CHUNK, step=LANES, unroll=8)
        def gbody(i):
          iv = idx_v[pl.ds(c * IDX_CHUNK + i, LANES)]
          out_v[buf, pl.ds(i, LANES)] = plsc.load_gather(row_v, [iv])

        pending[buf] = pltpu.async_copy(
            out_v.at[buf], out_hbm.at[j, pl.ds(c * IDX_CHUNK, IDX_CHUNK)],
            osem)
    for b in range(2):
      if pending[b] is not None:
        pending[b].wait()

  return gather


_sc_gather = _make_gather()

BLK = 2048


def _mlp_body(embt, w1t, b1, w2t, b2, w3t, b3, out):
  h = jnp.maximum(
      jnp.dot(w1t[...], embt[...], preferred_element_type=jnp.float32)
      + b1[...], 0.0)
  h = jnp.maximum(
      jnp.dot(w2t[...], h, preferred_element_type=jnp.float32) + b2[...], 0.0)
  out[...] = jnp.dot(w3t[...], h, preferred_element_type=jnp.float32) + b3[...]


def _mlp_t(embt, W1, b1, W2, b2, W3, b3):
  grid = (BATCH // BLK,)
  full = lambda shape: pl.BlockSpec(shape, lambda i: (0, 0))
  return pl.pallas_call(
      _mlp_body,
      grid=grid,
      in_specs=[
          pl.BlockSpec((EMBED, BLK), lambda i: (0, i)),
          full((H1, EMBED)),
          full((H1, 1)),
          full((H2, H1)),
          full((H2, 1)),
          full((H3, H2)),
          full((H3, 1)),
      ],
      out_specs=pl.BlockSpec((H3, BLK), lambda i: (0, i)),
      out_shape=jax.ShapeDtypeStruct((H3, BATCH), jnp.float32),
  )(embt, W1.T, b1.reshape(H1, 1), W2.T, b2.reshape(H2, 1), W3.T,
    b3.reshape(H3, 1))


@jax.jit
def kernel(books, table, W1, b1, W2, b2, W3, b3):
  embt = _sc_gather(table.T, books)
  outt = _mlp_t(embt, W1, b1, W2, b2, W3, b3)
  return outt.T


# speedup vs baseline: 2.5146x; 1.0600x over previous
import functools

import jax
import jax.numpy as jnp
from jax import lax
from jax.experimental import pallas as pl
from jax.experimental.pallas import tpu as pltpu
from jax.experimental.pallas import tpu_sc as plsc

BATCH = 16384
VOCAB = 100001
EMBED = 64
H1, H2, H3 = 128, 64, 32

NUM_CORES = 2
NUM_SUBCORES = 16
NUM_WORKERS = NUM_CORES * NUM_SUBCORES
ROWS_PER_TILE = EMBED // NUM_WORKERS
IDX_CHUNK = 4096
LANES = 16


def _make_gather():
  mesh = plsc.VectorSubcoreMesh(
      core_axis_name="c", subcore_axis_name="s",
      num_cores=NUM_CORES, num_subcores=NUM_SUBCORES)

  @functools.partial(
      pl.kernel,
      mesh=mesh,
      compiler_params=pltpu.CompilerParams(
          use_tc_tiling_on_sc=True, needs_layout_passes=False),
      out_type=jax.ShapeDtypeStruct((EMBED, BATCH), jnp.float32),
      scratch_types=[
          pltpu.VMEM((VOCAB,), jnp.float32),
          pltpu.VMEM((BATCH,), jnp.int32),
          pltpu.VMEM((2, IDX_CHUNK), jnp.float32),
          pltpu.SemaphoreType.DMA,
      ],
  )
  def gather(tablet_hbm, idx_hbm, out_hbm, row_v, idx_v, out_v, osem):
    tid = lax.axis_index("s") * NUM_CORES + lax.axis_index("c")
    pltpu.sync_copy(idx_hbm, idx_v)
    pending = [None, None]
    for p in range(ROWS_PER_TILE):
      j = tid * ROWS_PER_TILE + p
      pltpu.sync_copy(tablet_hbm.at[j], row_v)
      for c in range(BATCH // IDX_CHUNK):
        buf = c % 2
        if pending[buf] is not None:
          pending[buf].wait()

        @plsc.parallel_loop(0, IDX_CHUNK, step=LANES, unroll=8)
        def gbody(i):
          iv = idx_v[pl.ds(c * IDX_CHUNK + i, LANES)]
          out_v[buf, pl.ds(i, LANES)] = plsc.load_gather(row_v, [iv])

        pending[buf] = pltpu.async_copy(
            out_v.at[buf], out_hbm.at[j, pl.ds(c * IDX_CHUNK, IDX_CHUNK)],
            osem)
    for b in range(2):
      if pending[b] is not None:
        pending[b].wait()

  return gather


_sc_gather = _make_gather()

BLK = 4096


def _mlp_body(embt, w1t, b1, w2t, b2, w3t, b3, out):
  h = jnp.maximum(
      jnp.dot(w1t[...], embt[...], preferred_element_type=jnp.float32)
      + b1[...], 0.0)
  h = jnp.maximum(
      jnp.dot(w2t[...], h, preferred_element_type=jnp.float32) + b2[...], 0.0)
  out[...] = jnp.dot(w3t[...], h, preferred_element_type=jnp.float32) + b3[...]


def _mlp_t(embt, W1, b1, W2, b2, W3, b3):
  grid = (BATCH // BLK,)
  full = lambda shape: pl.BlockSpec(shape, lambda i: (0, 0))
  return pl.pallas_call(
      _mlp_body,
      grid=grid,
      in_specs=[
          pl.BlockSpec((EMBED, BLK), lambda i: (0, i)),
          full((H1, EMBED)),
          full((H1, 1)),
          full((H2, H1)),
          full((H2, 1)),
          full((H3, H2)),
          full((H3, 1)),
      ],
      out_specs=pl.BlockSpec((H3, BLK), lambda i: (0, i)),
      out_shape=jax.ShapeDtypeStruct((H3, BATCH), jnp.float32),
  )(embt, W1.T, b1.reshape(H1, 1), W2.T, b2.reshape(H2, 1), W3.T,
    b3.reshape(H3, 1))


@jax.jit
def kernel(books, table, W1, b1, W2, b2, W3, b3):
  embt = _sc_gather(table.T, books)
  outt = _mlp_t(embt, W1, b1, W2, b2, W3, b3)
  return outt.T


# speedup vs baseline: 2.5893x; 1.0297x over previous
import functools

import jax
import jax.numpy as jnp
from jax import lax
from jax.experimental import pallas as pl
from jax.experimental.pallas import tpu as pltpu
from jax.experimental.pallas import tpu_sc as plsc

BATCH = 16384
VOCAB = 100001
EMBED = 64
H1, H2, H3 = 128, 64, 32

NUM_CORES = 2
NUM_SUBCORES = 16
NUM_WORKERS = NUM_CORES * NUM_SUBCORES
ROWS_PER_TILE = EMBED // NUM_WORKERS
IDX_CHUNK = 4096
LANES = 16


def _make_gather():
  mesh = plsc.VectorSubcoreMesh(
      core_axis_name="c", subcore_axis_name="s",
      num_cores=NUM_CORES, num_subcores=NUM_SUBCORES)

  @functools.partial(
      pl.kernel,
      mesh=mesh,
      compiler_params=pltpu.CompilerParams(
          use_tc_tiling_on_sc=True, needs_layout_passes=False),
      out_type=jax.ShapeDtypeStruct((EMBED, BATCH), jnp.float32),
      scratch_types=[
          pltpu.VMEM((VOCAB,), jnp.float32),
          pltpu.VMEM((BATCH,), jnp.int32),
          pltpu.VMEM((2, IDX_CHUNK), jnp.float32),
          pltpu.SemaphoreType.DMA,
      ],
  )
  def gather(tablet_hbm, idx_hbm, out_hbm, row_v, idx_v, out_v, osem):
    tid = lax.axis_index("s") * NUM_CORES + lax.axis_index("c")
    pltpu.sync_copy(idx_hbm, idx_v)
    pending = [None, None]
    for p in range(ROWS_PER_TILE):
      j = tid * ROWS_PER_TILE + p
      pltpu.sync_copy(tablet_hbm.at[j], row_v)
      for c in range(BATCH // IDX_CHUNK):
        buf = c % 2
        if pending[buf] is not None:
          pending[buf].wait()

        @plsc.parallel_loop(0, IDX_CHUNK, step=LANES, unroll=8)
        def gbody(i):
          iv = idx_v[pl.ds(c * IDX_CHUNK + i, LANES)]
          out_v[buf, pl.ds(i, LANES)] = plsc.load_gather(row_v, [iv])

        pending[buf] = pltpu.async_copy(
            out_v.at[buf], out_hbm.at[j, pl.ds(c * IDX_CHUNK, IDX_CHUNK)],
            osem)
    for b in range(2):
      if pending[b] is not None:
        pending[b].wait()

  return gather


_sc_gather = _make_gather()

BLK = 8192


def _mlp_body(embt, w1t, b1, w2t, b2, w3t, b3, out):
  h = jnp.maximum(
      jnp.dot(w1t[...], embt[...], preferred_element_type=jnp.float32)
      + b1[...], 0.0)
  h = jnp.maximum(
      jnp.dot(w2t[...], h, preferred_element_type=jnp.float32) + b2[...], 0.0)
  out[...] = jnp.dot(w3t[...], h, preferred_element_type=jnp.float32) + b3[...]


def _mlp_t(embt, W1, b1, W2, b2, W3, b3):
  grid = (BATCH // BLK,)
  full = lambda shape: pl.BlockSpec(shape, lambda i: (0, 0))
  return pl.pallas_call(
      _mlp_body,
      grid=grid,
      in_specs=[
          pl.BlockSpec((EMBED, BLK), lambda i: (0, i)),
          full((H1, EMBED)),
          full((H1, 1)),
          full((H2, H1)),
          full((H2, 1)),
          full((H3, H2)),
          full((H3, 1)),
      ],
      out_specs=pl.BlockSpec((H3, BLK), lambda i: (0, i)),
      out_shape=jax.ShapeDtypeStruct((H3, BATCH), jnp.float32),
  )(embt, W1.T, b1.reshape(H1, 1), W2.T, b2.reshape(H2, 1), W3.T,
    b3.reshape(H3, 1))


@jax.jit
def kernel(books, table, W1, b1, W2, b2, W3, b3):
  embt = _sc_gather(table.T, books)
  outt = _mlp_t(embt, W1, b1, W2, b2, W3, b3)
  return outt.T
